# speedup vs baseline: 14.8575x; 14.8575x over previous
import jax
import jax.numpy as jnp
from jax.experimental import pallas as pl

_K = 256
_ROWS_PER_BLOCK = 8


def _block_body(x_ref, o_ref):
    xb = x_ref[...]
    u = jax.lax.bitcast_convert_type(xb, jnp.int32) & jnp.int32(0x7FFFFFFF)

    def bit_body(i, t):
        cand = t | (jnp.int32(1) << (jnp.int32(30) - i))
        cnt = jnp.sum((u >= cand).astype(jnp.int32), axis=1, keepdims=True)
        return jnp.where(cnt >= _K, cand, t)

    t0 = jnp.zeros((x_ref.shape[0], 1), jnp.int32)
    t = jax.lax.fori_loop(0, 31, bit_body, t0)
    o_ref[...] = jnp.where(u >= t, xb, jnp.float32(0.0))


def kernel(x, interpret=False):
    m, n = x.shape
    r = _ROWS_PER_BLOCK
    return pl.pallas_call(
        _block_body,
        grid=(m // r,),
        in_specs=[pl.BlockSpec((r, n), lambda i: (i, 0))],
        out_specs=pl.BlockSpec((r, n), lambda i: (i, 0)),
        out_shape=jax.ShapeDtypeStruct(x.shape, x.dtype),
        interpret=interpret,
    )(x)


# speedup vs baseline: 23.3068x; 1.5687x over previous
import jax
import jax.numpy as jnp
from jax.experimental import pallas as pl

_K = 256
_ROWS_PER_BLOCK = 8


def _block_body(x_ref, o_ref):
    r, n = x_ref.shape
    xb = x_ref[...]
    u = jax.lax.bitcast_convert_type(xb, jnp.int32) & jnp.int32(0x7FFFFFFF)
    u3 = u.reshape(r, n // 128, 128)

    def _counts3(t3, sh):
        c1 = t3 | (jnp.int32(1) << sh)
        c2 = t3 | (jnp.int32(2) << sh)
        c3 = t3 | (jnp.int32(3) << sh)
        f = (
            (u3 >= c1).astype(jnp.int32)
            + jnp.where(u3 >= c2, jnp.int32(1 << 10), 0)
            + jnp.where(u3 >= c3, jnp.int32(1 << 20), 0)
        )
        s = jnp.sum(f, axis=1)
        cnt1 = jnp.sum(s & 1023, axis=-1).reshape(r, 1, 1)
        cnt2 = jnp.sum((s >> 10) & 1023, axis=-1).reshape(r, 1, 1)
        cnt3 = jnp.sum(s >> 20, axis=-1).reshape(r, 1, 1)
        return c1, c2, c3, cnt1, cnt2, cnt3

    def phase(i, t3):
        sh = jnp.int32(29) - 2 * i
        c1, c2, c3, cnt1, cnt2, cnt3 = _counts3(t3, sh)
        t3 = jnp.where(
            cnt3 >= _K,
            c3,
            jnp.where(cnt2 >= _K, c2, jnp.where(cnt1 >= _K, c1, t3)),
        )
        return t3

    t3 = jnp.zeros((r, 1, 1), jnp.int32)
    t3 = jax.lax.fori_loop(0, 15, phase, t3)
    cand = t3 | jnp.int32(1)
    cnt = jnp.sum((u3 >= cand).astype(jnp.int32), axis=(1, 2)).reshape(r, 1, 1)
    t3 = jnp.where(cnt >= _K, cand, t3)
    t = t3.reshape(r, 1)
    o_ref[...] = jnp.where(u >= t, xb, jnp.float32(0.0))


def kernel(x, interpret=False):
    m, n = x.shape
    r = _ROWS_PER_BLOCK
    return pl.pallas_call(
        _block_body,
        grid=(m // r,),
        in_specs=[pl.BlockSpec((r, n), lambda i: (i, 0))],
        out_specs=pl.BlockSpec((r, n), lambda i: (i, 0)),
        out_shape=jax.ShapeDtypeStruct(x.shape, x.dtype),
        interpret=interpret,
    )(x)


# speedup vs baseline: 25.5006x; 1.0941x over previous
import jax
import jax.numpy as jnp
from jax.experimental import pallas as pl

_K = 256
_ROWS_PER_BLOCK = 16


def _block_body(x_ref, o_ref):
    r, n = x_ref.shape
    xb = x_ref[...]
    u = jax.lax.bitcast_convert_type(xb, jnp.int32) & jnp.int32(0x7FFFFFFF)
    u3 = u.reshape(r, n // 128, 128)

    def _counts3(t3, sh):
        c1 = t3 | (jnp.int32(1) << sh)
        c2 = t3 | (jnp.int32(2) << sh)
        c3 = t3 | (jnp.int32(3) << sh)
        f = (
            (u3 >= c1).astype(jnp.int32)
            + jnp.where(u3 >= c2, jnp.int32(1 << 10), 0)
            + jnp.where(u3 >= c3, jnp.int32(1 << 20), 0)
        )
        s = jnp.sum(f, axis=1)
        cnt1 = jnp.sum(s & 1023, axis=-1).reshape(r, 1, 1)
        cnt2 = jnp.sum((s >> 10) & 1023, axis=-1).reshape(r, 1, 1)
        cnt3 = jnp.sum(s >> 20, axis=-1).reshape(r, 1, 1)
        return c1, c2, c3, cnt1, cnt2, cnt3

    def phase(i, t3):
        sh = jnp.int32(29) - 2 * i
        c1, c2, c3, cnt1, cnt2, cnt3 = _counts3(t3, sh)
        t3 = jnp.where(
            cnt3 >= _K,
            c3,
            jnp.where(cnt2 >= _K, c2, jnp.where(cnt1 >= _K, c1, t3)),
        )
        return t3

    t3 = jnp.zeros((r, 1, 1), jnp.int32)
    t3 = jax.lax.fori_loop(0, 15, phase, t3)
    cand = t3 | jnp.int32(1)
    cnt = jnp.sum((u3 >= cand).astype(jnp.int32), axis=(1, 2)).reshape(r, 1, 1)
    t3 = jnp.where(cnt >= _K, cand, t3)
    t = t3.reshape(r, 1)
    o_ref[...] = jnp.where(u >= t, xb, jnp.float32(0.0))


def kernel(x, interpret=False):
    m, n = x.shape
    r = _ROWS_PER_BLOCK
    return pl.pallas_call(
        _block_body,
        grid=(m // r,),
        in_specs=[pl.BlockSpec((r, n), lambda i: (i, 0))],
        out_specs=pl.BlockSpec((r, n), lambda i: (i, 0)),
        out_shape=jax.ShapeDtypeStruct(x.shape, x.dtype),
        interpret=interpret,
    )(x)


# speedup vs baseline: 26.5404x; 1.0408x over previous
import jax
import jax.numpy as jnp
from jax.experimental import pallas as pl

_K = 256
_ROWS_PER_BLOCK = 32


def _block_body(x_ref, o_ref):
    r, n = x_ref.shape
    xb = x_ref[...]
    u = jax.lax.bitcast_convert_type(xb, jnp.int32) & jnp.int32(0x7FFFFFFF)
    u3 = u.reshape(r, n // 128, 128)

    def _counts3(t3, sh):
        c1 = t3 | (jnp.int32(1) << sh)
        c2 = t3 | (jnp.int32(2) << sh)
        c3 = t3 | (jnp.int32(3) << sh)
        f = (
            (u3 >= c1).astype(jnp.int32)
            + jnp.where(u3 >= c2, jnp.int32(1 << 10), 0)
            + jnp.where(u3 >= c3, jnp.int32(1 << 20), 0)
        )
        s = jnp.sum(f, axis=1)
        cnt1 = jnp.sum(s & 1023, axis=-1).reshape(r, 1, 1)
        cnt2 = jnp.sum((s >> 10) & 1023, axis=-1).reshape(r, 1, 1)
        cnt3 = jnp.sum(s >> 20, axis=-1).reshape(r, 1, 1)
        return c1, c2, c3, cnt1, cnt2, cnt3

    def phase(i, t3):
        sh = jnp.int32(29) - 2 * i
        c1, c2, c3, cnt1, cnt2, cnt3 = _counts3(t3, sh)
        t3 = jnp.where(
            cnt3 >= _K,
            c3,
            jnp.where(cnt2 >= _K, c2, jnp.where(cnt1 >= _K, c1, t3)),
        )
        return t3

    t3 = jnp.zeros((r, 1, 1), jnp.int32)
    t3 = jax.lax.fori_loop(0, 15, phase, t3)
    cand = t3 | jnp.int32(1)
    cnt = jnp.sum((u3 >= cand).astype(jnp.int32), axis=(1, 2)).reshape(r, 1, 1)
    t3 = jnp.where(cnt >= _K, cand, t3)
    t = t3.reshape(r, 1)
    o_ref[...] = jnp.where(u >= t, xb, jnp.float32(0.0))


def kernel(x, interpret=False):
    m, n = x.shape
    r = _ROWS_PER_BLOCK
    return pl.pallas_call(
        _block_body,
        grid=(m // r,),
        in_specs=[pl.BlockSpec((r, n), lambda i: (i, 0))],
        out_specs=pl.BlockSpec((r, n), lambda i: (i, 0)),
        out_shape=jax.ShapeDtypeStruct(x.shape, x.dtype),
        interpret=interpret,
    )(x)


# speedup vs baseline: 26.6300x; 1.0034x over previous
import jax
import jax.numpy as jnp
from jax.experimental import pallas as pl

_K = 256
_ROWS_PER_BLOCK = 64


def _block_body(x_ref, o_ref):
    r, n = x_ref.shape
    xb = x_ref[...]
    u = jax.lax.bitcast_convert_type(xb, jnp.int32) & jnp.int32(0x7FFFFFFF)
    u3 = u.reshape(r, n // 128, 128)

    def _counts3(t3, sh):
        c1 = t3 | (jnp.int32(1) << sh)
        c2 = t3 | (jnp.int32(2) << sh)
        c3 = t3 | (jnp.int32(3) << sh)
        f = (
            (u3 >= c1).astype(jnp.int32)
            + jnp.where(u3 >= c2, jnp.int32(1 << 10), 0)
            + jnp.where(u3 >= c3, jnp.int32(1 << 20), 0)
        )
        s = jnp.sum(f, axis=1)
        cnt1 = jnp.sum(s & 1023, axis=-1).reshape(r, 1, 1)
        cnt2 = jnp.sum((s >> 10) & 1023, axis=-1).reshape(r, 1, 1)
        cnt3 = jnp.sum(s >> 20, axis=-1).reshape(r, 1, 1)
        return c1, c2, c3, cnt1, cnt2, cnt3

    def phase(i, t3):
        sh = jnp.int32(29) - 2 * i
        c1, c2, c3, cnt1, cnt2, cnt3 = _counts3(t3, sh)
        t3 = jnp.where(
            cnt3 >= _K,
            c3,
            jnp.where(cnt2 >= _K, c2, jnp.where(cnt1 >= _K, c1, t3)),
        )
        return t3

    t3 = jnp.zeros((r, 1, 1), jnp.int32)
    t3 = jax.lax.fori_loop(0, 15, phase, t3)
    cand = t3 | jnp.int32(1)
    cnt = jnp.sum((u3 >= cand).astype(jnp.int32), axis=(1, 2)).reshape(r, 1, 1)
    t3 = jnp.where(cnt >= _K, cand, t3)
    t = t3.reshape(r, 1)
    o_ref[...] = jnp.where(u >= t, xb, jnp.float32(0.0))


def kernel(x, interpret=False):
    m, n = x.shape
    r = _ROWS_PER_BLOCK
    return pl.pallas_call(
        _block_body,
        grid=(m // r,),
        in_specs=[pl.BlockSpec((r, n), lambda i: (i, 0))],
        out_specs=pl.BlockSpec((r, n), lambda i: (i, 0)),
        out_shape=jax.ShapeDtypeStruct(x.shape, x.dtype),
        interpret=interpret,
    )(x)
